# baseline (speedup 1.0000x reference)
import functools

import jax
import jax.numpy as jnp
from jax import lax
from jax.experimental import pallas as pl
from jax.experimental.pallas import tpu as pltpu
from jax.experimental.pallas import tpu_sc as plsc

N = 100000
DIM = 256
DG, DPL = 128, 128
R = 128
NB = (N + R - 1) // R
NW = 32
BPW = 25
SLAB = BPW * R
L = 16
U = 4


def _body(g_h, pl_h, wg_h, wpl_h, out_h,
          idx_g, idx_pl, rows0, rows1, wg_v, wpl_v,
          sem_i, sw0, sw1):
    c = lax.axis_index("c")
    s = lax.axis_index("s")
    w = s * 2 + c
    start = (w * (NB - BPW)) // (NW - 1)
    e0 = start * R

    hs = [
        pltpu.async_copy(g_h.at[pl.ds(e0, SLAB)], idx_g, sem_i),
        pltpu.async_copy(pl_h.at[pl.ds(e0, SLAB)], idx_pl, sem_i),
        pltpu.async_copy(wg_h, wg_v, sem_i),
        pltpu.async_copy(wpl_h, wpl_v, sem_i),
    ]
    for h in hs:
        h.wait()

    iota = lax.iota(jnp.int32, L)
    colc = [m * L + iota for m in range(DIM // L)]

    bufs = (rows0, rows1)
    sws = (sw0, sw1)

    def fill_block(j25):
        buf = bufs[j25 % 2]
        base = jnp.full((L,), j25 * R, jnp.int32)

        def group(i, carry):
            n0 = i * U
            for k in range(U):
                n = n0 + k
                nv = base + n
                gb = plsc.load_gather(idx_g, [nv])
                pb = plsc.load_gather(idx_pl, [nv])
                vsg = [plsc.load_gather(wg_v, [gb, colc[m]])
                       for m in range(DG // L)]
                vsp = [plsc.load_gather(wpl_v, [pb, colc[m]])
                       for m in range(DPL // L)]
                for m in range(DG // L):
                    buf[n, pl.ds(m * L, L)] = vsg[m]
                for m in range(DPL // L):
                    buf[n, pl.ds(DG + m * L, L)] = vsp[m]
            return carry

        lax.fori_loop(0, R // U, group, 0)

    def fire_write(j):
        slot = j % 2
        base = jnp.minimum((start + j) * R, N - R)
        return pltpu.async_copy(bufs[slot], out_h.at[pl.ds(base, R), :],
                                sws[slot])

    wh = [None] * BPW
    for j in range(BPW):
        if j >= 2:
            wh[j - 2].wait()
        fill_block(j)
        wh[j] = fire_write(j)
    wh[BPW - 2].wait()
    wh[BPW - 1].wait()


@jax.jit
def kernel(group, period, ls, Wg, Wp, Wl):
    Wpl = jnp.concatenate(
        [jnp.repeat(Wp, 3, axis=0), jnp.tile(Wl, (7, 1))], axis=1)

    def layout(x):
        return jnp.concatenate([x[:(NB - 1) * R], x[N - R:]])

    g1 = layout(group.astype(jnp.int32))
    pl1 = layout(period.astype(jnp.int32) * 3 + ls.astype(jnp.int32))

    mesh = plsc.VectorSubcoreMesh(core_axis_name="c", subcore_axis_name="s")
    run = functools.partial(
        pl.kernel,
        mesh=mesh,
        compiler_params=pltpu.CompilerParams(needs_layout_passes=False),
        out_type=jax.ShapeDtypeStruct((N, DIM), jnp.float32),
        scratch_types=[
            pltpu.VMEM((SLAB,), jnp.int32),
            pltpu.VMEM((SLAB,), jnp.int32),
            pltpu.VMEM((R, DIM), jnp.float32),
            pltpu.VMEM((R, DIM), jnp.float32),
            pltpu.VMEM((18, DG), jnp.float32),
            pltpu.VMEM((21, DPL), jnp.float32),
            pltpu.SemaphoreType.DMA,
            pltpu.SemaphoreType.DMA,
            pltpu.SemaphoreType.DMA,
        ],
    )(_body)
    return run(g1, pl1, Wg, Wpl)
